# baseline (speedup 1.0000x reference)
import functools

import jax
import jax.numpy as jnp
from jax.experimental import pallas as pl
from jax.experimental.pallas import tpu as pltpu


def _block_kernel(x1u_ref, x2_ref, w1_ref, b1_ref, w2_ref, b2_ref, o_ref,
                  *, th, wu):
    c2 = x2_ref.shape[1]

    s = pl.program_id(1)

    def lane_iota(shape):
        return jax.lax.broadcasted_iota(jnp.int32, shape, len(shape) - 1) % wu

    u = x1u_ref[0, :, pl.ds(s * th * wu, (th + 4) * wu)]

    lane = lane_iota(u.shape)
    um1 = jnp.where(lane == 0, u,
                    jnp.concatenate([u[:, -1:], u[:, :-1]], axis=1))
    up1 = jnp.where(lane == wu - 1, u,
                    jnp.concatenate([u[:, 1:], u[:, -1:]], axis=1))

    def conv(taps, w_ref, rows):
        acc = None
        for ky in range(3):
            for kx in range(3):
                a = taps[kx][:, ky * wu:(ky + rows) * wu]
                p = jnp.dot(w_ref[ky, kx], a,
                            preferred_element_type=jnp.float32)
                acc = p if acc is None else acc + p
        return acc

    y1 = conv((um1, u, up1), w1_ref, th + 2)

    first = s == 0
    last = s == pl.num_programs(1) - 1
    y1 = jnp.where(first,
                   jnp.concatenate([y1[:, 2 * wu:3 * wu], y1[:, wu:]], axis=1),
                   y1)
    y1 = jnp.where(last,
                   jnp.concatenate([y1[:, :(th + 1) * wu],
                                    y1[:, (th - 1) * wu:th * wu]], axis=1),
                   y1)

    a1 = jnp.maximum(y1 + b1_ref[...], 0.0).astype(jnp.bfloat16)

    lane1 = lane_iota(a1.shape)
    rm1 = jnp.concatenate([a1[:, -1:], a1[:, :-1]], axis=1)
    rp1 = jnp.concatenate([a1[:, 1:], a1[:, -1:]], axis=1)
    am1 = jnp.where(lane1 == 0, rp1, rm1)
    ap1 = jnp.where(lane1 == wu - 1, rm1, rp1)

    y2 = conv((am1, a1, ap1), w2_ref, th)
    y2 = jnp.maximum(y2 + b2_ref[...], 0.0)

    o_ref[0, :c2, :] = x2_ref[0]
    o_ref[0, c2:, :] = y2.astype(o_ref.dtype)


def _tile_rows(hu, cap=32):
    th = min(hu, cap)
    while hu % th != 0 or th % 2 != 0:
        th -= 1
    return max(th, 2)


def kernel(x1_nchw, x2_nchw, w1, b1, w2, b2):
    n, cin, hs, ws = x1_nchw.shape
    cout = w1.shape[-1]
    c2 = x2_nchw.shape[1]
    hu, wu = 2 * hs, 2 * ws

    th = _tile_rows(hu)
    n_strips = hu // th

    x1u = jnp.repeat(jnp.repeat(x1_nchw.astype(jnp.bfloat16), 2, axis=2),
                     2, axis=3)
    x1u = jnp.pad(x1u, ((0, 0), (0, 0), (2, 2), (0, 0)), mode="edge")
    x1u = x1u.reshape(n, cin, (hu + 4) * wu)
    x2f = x2_nchw.reshape(n, c2, hu * wu)

    w1t = jnp.transpose(w1, (0, 1, 3, 2)).astype(jnp.bfloat16)
    w2t = jnp.transpose(w2, (0, 1, 3, 2)).astype(jnp.bfloat16)
    b1c = b1.reshape(cout, 1)
    b2c = b2.reshape(cout, 1)

    body = functools.partial(_block_kernel, th=th, wu=wu)

    out_flat = pl.pallas_call(
        body,
        out_shape=jax.ShapeDtypeStruct((n, c2 + cout, hu * wu),
                                       x1_nchw.dtype),
        grid=(n, n_strips),
        in_specs=[
            pl.BlockSpec((1, cin, (hu + 4) * wu), lambda b, s: (b, 0, 0)),
            pl.BlockSpec((1, c2, th * wu), lambda b, s: (b, 0, s)),
            pl.BlockSpec((3, 3, cout, cin), lambda b, s: (0, 0, 0, 0)),
            pl.BlockSpec((cout, 1), lambda b, s: (0, 0)),
            pl.BlockSpec((3, 3, cout, cout), lambda b, s: (0, 0, 0, 0)),
            pl.BlockSpec((cout, 1), lambda b, s: (0, 0)),
        ],
        out_specs=pl.BlockSpec((1, c2 + cout, th * wu),
                               lambda b, s: (b, 0, s)),
        compiler_params=pltpu.CompilerParams(
            dimension_semantics=("parallel", "parallel"),
            vmem_limit_bytes=48 * 2 ** 20),
    )(x1u, x2f, w1t, b1c, w2t, b2c)

    return out_flat.reshape(n, c2 + cout, hu, wu)


# speedup vs baseline: 1.6625x; 1.6625x over previous
import functools

import jax
import jax.numpy as jnp
from jax.experimental import pallas as pl
from jax.experimental.pallas import tpu as pltpu


def _block_kernel(x1e_ref, x2_ref, w1a_ref, w1b_ref, b1_ref, w2_ref, b2_ref,
                  o_ref, *, th, ws):
    cin = x1e_ref.shape[1]
    c2 = x2_ref.shape[1]
    cout = b1_ref.shape[1]

    s = pl.program_id(1)
    nlow = th // 2 + 2

    def eye(n):
        r = jax.lax.broadcasted_iota(jnp.int32, (n, n), 0)
        c = jax.lax.broadcasted_iota(jnp.int32, (n, n), 1)
        return (r == c).astype(jnp.bfloat16)

    xs = x1e_ref[0, :, pl.ds(s * (th // 2) * ws, nlow * ws)]
    low = jax.lax.dot_general(xs, eye(cin), (((0,), (0,)), ((), ())),
                              preferred_element_type=jnp.float32)
    low = low.astype(jnp.bfloat16).reshape(nlow, ws, cin)

    uph = jnp.concatenate([low[:, None], low[:, None]],
                          axis=1).reshape(2 * nlow, ws, cin)

    uphm1 = jnp.concatenate([uph[:, :1], uph[:, :ws - 1]], axis=1)
    uphp1 = jnp.concatenate([uph[:, 1:], uph[:, ws - 1:]], axis=1)

    ops1a = jnp.concatenate([uphm1, uph], axis=2)
    ops1b = jnp.concatenate([uph, uphp1], axis=2)

    def conv3(ops, w_ref, rows, k):
        acc = None
        for ky in range(3):
            a = ops[ky:ky + rows].reshape(rows * ws, k)
            p = jnp.dot(a, w_ref[ky], preferred_element_type=jnp.float32)
            acc = p if acc is None else acc + p
        return acc.reshape(rows, ws, -1)

    y1a = conv3(ops1a, w1a_ref, th + 2, 2 * cin)
    y1b = conv3(ops1b, w1b_ref, th + 2, 2 * cin)

    first = s == 0
    last = s == pl.num_programs(1) - 1
    y1a = jnp.where(first, jnp.concatenate([y1a[2:3], y1a[1:]], axis=0), y1a)
    y1b = jnp.where(first, jnp.concatenate([y1b[2:3], y1b[1:]], axis=0), y1b)
    y1a = jnp.where(last, jnp.concatenate([y1a[:th + 1], y1a[th - 1:th]],
                                          axis=0), y1a)
    y1b = jnp.where(last, jnp.concatenate([y1b[:th + 1], y1b[th - 1:th]],
                                          axis=0), y1b)

    b1v = b1_ref[...]
    a1a = jnp.maximum(y1a + b1v, 0.0).astype(jnp.bfloat16)
    a1b = jnp.maximum(y1b + b1v, 0.0).astype(jnp.bfloat16)

    a1bm1 = jnp.concatenate([a1b[:, :1], a1b[:, :ws - 1]], axis=1)
    a1ap1 = jnp.concatenate([a1a[:, 1:], a1a[:, ws - 1:]], axis=1)
    ops2a = jnp.concatenate([a1bm1, a1a, a1b], axis=2)
    ops2b = jnp.concatenate([a1a, a1b, a1ap1], axis=2)

    y2a = conv3(ops2a, w2_ref, th, 3 * cout)
    y2b = conv3(ops2b, w2_ref, th, 3 * cout)

    b2v = b2_ref[...]
    y2a = jnp.maximum(y2a + b2v, 0.0)
    y2b = jnp.maximum(y2b + b2v, 0.0)

    y2 = jnp.concatenate([y2a[:, :, None], y2b[:, :, None]],
                         axis=2).reshape(th * 2 * ws, cout)

    hi = y2.astype(jnp.bfloat16)
    lo = (y2 - hi.astype(jnp.float32)).astype(jnp.bfloat16)
    ic = eye(cout)
    dn = (((1,), (1,)), ((), ()))
    y2t = (jax.lax.dot_general(ic, hi, dn,
                               preferred_element_type=jnp.float32)
           + jax.lax.dot_general(ic, lo, dn,
                                 preferred_element_type=jnp.float32))

    o_ref[0, :c2, :] = x2_ref[0]
    o_ref[0, c2:, :] = y2t.astype(o_ref.dtype)


def _tile_rows(hu, cap=32):
    th = min(hu, cap)
    while hu % th != 0 or th % 2 != 0:
        th -= 1
    return max(th, 2)


def kernel(x1_nchw, x2_nchw, w1, b1, w2, b2):
    n, cin, hs, ws = x1_nchw.shape
    cout = w1.shape[-1]
    c2 = x2_nchw.shape[1]
    hu, wu = 2 * hs, 2 * ws

    th = _tile_rows(hu)
    n_strips = hu // th

    x1e = jnp.pad(x1_nchw.astype(jnp.bfloat16),
                  ((0, 0), (0, 0), (1, 1), (0, 0)), mode="edge")
    x1e = x1e.reshape(n, cin, (hs + 2) * ws)
    x2f = x2_nchw.reshape(n, c2, hu * wu)

    w1a = jnp.concatenate([w1[:, 0], w1[:, 1] + w1[:, 2]],
                          axis=1).astype(jnp.bfloat16)
    w1b = jnp.concatenate([w1[:, 0] + w1[:, 1], w1[:, 2]],
                          axis=1).astype(jnp.bfloat16)
    w2s = w2.reshape(3, 3 * cout, cout).astype(jnp.bfloat16)
    b1r = b1.reshape(1, cout)
    b2r = b2.reshape(1, cout)

    body = functools.partial(_block_kernel, th=th, ws=ws)

    out_flat = pl.pallas_call(
        body,
        out_shape=jax.ShapeDtypeStruct((n, c2 + cout, hu * wu),
                                       x1_nchw.dtype),
        grid=(n, n_strips),
        in_specs=[
            pl.BlockSpec((1, cin, (hs + 2) * ws), lambda b, s: (b, 0, 0)),
            pl.BlockSpec((1, c2, th * wu), lambda b, s: (b, 0, s)),
            pl.BlockSpec((3, 2 * cin, cout), lambda b, s: (0, 0, 0)),
            pl.BlockSpec((3, 2 * cin, cout), lambda b, s: (0, 0, 0)),
            pl.BlockSpec((1, cout), lambda b, s: (0, 0)),
            pl.BlockSpec((3, 3 * cout, cout), lambda b, s: (0, 0, 0)),
            pl.BlockSpec((1, cout), lambda b, s: (0, 0)),
        ],
        out_specs=pl.BlockSpec((1, c2 + cout, th * wu),
                               lambda b, s: (b, 0, s)),
        compiler_params=pltpu.CompilerParams(
            dimension_semantics=("parallel", "parallel"),
            vmem_limit_bytes=48 * 2 ** 20),
    )(x1e, x2f, w1a, w1b, b1r, w2s, b2r)

    return out_flat.reshape(n, c2 + cout, hu, wu)


# speedup vs baseline: 2.7431x; 1.6499x over previous
import functools

import jax
import jax.numpy as jnp
from jax.experimental import pallas as pl
from jax.experimental.pallas import tpu as pltpu


def _block_kernel(x1e_ref, x2_ref, w1a_ref, w1b_ref, b1_ref, w2_ref, b2_ref,
                  o_ref, *, th, ws):
    cin = x1e_ref.shape[1]
    c2 = x2_ref.shape[1]
    cout = b1_ref.shape[1]

    s = pl.program_id(1)
    nlow = th // 2 + 2

    def eye(n):
        r = jax.lax.broadcasted_iota(jnp.int32, (n, n), 0)
        c = jax.lax.broadcasted_iota(jnp.int32, (n, n), 1)
        return (r == c).astype(jnp.bfloat16)

    xs = x1e_ref[0, :, pl.ds(s * (th // 2) * ws, nlow * ws)]
    low = jax.lax.dot_general(xs, eye(cin), (((0,), (0,)), ((), ())),
                              preferred_element_type=jnp.float32)
    low = low.astype(jnp.bfloat16).reshape(nlow, ws, cin)

    uph = jnp.concatenate([low[:, None], low[:, None]],
                          axis=1).reshape(2 * nlow, ws, cin)

    uphm1 = jnp.concatenate([uph[:, :1], uph[:, :ws - 1]], axis=1)
    uphp1 = jnp.concatenate([uph[:, 1:], uph[:, ws - 1:]], axis=1)

    ops1a = jnp.concatenate([uphm1, uph], axis=2)
    ops1b = jnp.concatenate([uph, uphp1], axis=2)

    def conv3(ops, w_ref, rows, k):
        acc = None
        for ky in range(3):
            a = ops[ky:ky + rows].reshape(rows * ws, k)
            p = jnp.dot(a, w_ref[ky], preferred_element_type=jnp.float32)
            acc = p if acc is None else acc + p
        return acc.reshape(rows, ws, -1)

    y1a = conv3(ops1a, w1a_ref, th + 2, 2 * cin)
    y1b = conv3(ops1b, w1b_ref, th + 2, 2 * cin)

    first = s == 0
    last = s == pl.num_programs(1) - 1
    y1a = jnp.where(first, jnp.concatenate([y1a[2:3], y1a[1:]], axis=0), y1a)
    y1b = jnp.where(first, jnp.concatenate([y1b[2:3], y1b[1:]], axis=0), y1b)
    y1a = jnp.where(last, jnp.concatenate([y1a[:th + 1], y1a[th - 1:th]],
                                          axis=0), y1a)
    y1b = jnp.where(last, jnp.concatenate([y1b[:th + 1], y1b[th - 1:th]],
                                          axis=0), y1b)

    b1v = b1_ref[...]
    a1a = jnp.maximum(y1a + b1v, 0.0).astype(jnp.bfloat16)
    a1b = jnp.maximum(y1b + b1v, 0.0).astype(jnp.bfloat16)

    a1bm1 = jnp.concatenate([a1b[:, :1], a1b[:, :ws - 1]], axis=1)
    a1ap1 = jnp.concatenate([a1a[:, 1:], a1a[:, ws - 1:]], axis=1)
    ops2a = jnp.concatenate([a1bm1, a1a, a1b], axis=2)
    ops2b = jnp.concatenate([a1a, a1b, a1ap1], axis=2)

    y2a = conv3(ops2a, w2_ref, th, 3 * cout)
    y2b = conv3(ops2b, w2_ref, th, 3 * cout)

    b2v = b2_ref[...]
    y2a = jnp.maximum(y2a + b2v, 0.0)
    y2b = jnp.maximum(y2b + b2v, 0.0)

    y2 = jnp.concatenate([y2a[:, :, None], y2b[:, :, None]],
                         axis=2).reshape(th * 2 * ws, cout)

    hi = y2.astype(jnp.bfloat16)
    lo = (y2 - hi.astype(jnp.float32)).astype(jnp.bfloat16)
    ic = eye(cout)
    dn = (((1,), (1,)), ((), ()))
    y2t = (jax.lax.dot_general(ic, hi, dn,
                               preferred_element_type=jnp.float32)
           + jax.lax.dot_general(ic, lo, dn,
                                 preferred_element_type=jnp.float32))

    o_ref[0, :c2] = x2_ref[0]
    o_ref[0, c2:] = y2t.reshape(cout, th, 2 * ws).astype(o_ref.dtype)


def _tile_rows(hu, cap=32):
    th = min(hu, cap)
    while hu % th != 0 or th % 2 != 0:
        th -= 1
    return max(th, 2)


def kernel(x1_nchw, x2_nchw, w1, b1, w2, b2):
    n, cin, hs, ws = x1_nchw.shape
    cout = w1.shape[-1]
    c2 = x2_nchw.shape[1]
    hu, wu = 2 * hs, 2 * ws

    th = _tile_rows(hu)
    n_strips = hu // th

    x1e = jnp.pad(x1_nchw.astype(jnp.bfloat16),
                  ((0, 0), (0, 0), (1, 1), (0, 0)), mode="edge")
    x1e = x1e.reshape(n, cin, (hs + 2) * ws)

    w1a = jnp.concatenate([w1[:, 0], w1[:, 1] + w1[:, 2]],
                          axis=1).astype(jnp.bfloat16)
    w1b = jnp.concatenate([w1[:, 0] + w1[:, 1], w1[:, 2]],
                          axis=1).astype(jnp.bfloat16)
    w2s = w2.reshape(3, 3 * cout, cout).astype(jnp.bfloat16)
    b1r = b1.reshape(1, cout)
    b2r = b2.reshape(1, cout)

    body = functools.partial(_block_kernel, th=th, ws=ws)

    out = pl.pallas_call(
        body,
        out_shape=jax.ShapeDtypeStruct((n, c2 + cout, hu, wu),
                                       x1_nchw.dtype),
        grid=(n, n_strips),
        in_specs=[
            pl.BlockSpec((1, cin, (hs + 2) * ws), lambda b, s: (b, 0, 0)),
            pl.BlockSpec((1, c2, th, wu), lambda b, s: (b, 0, s, 0)),
            pl.BlockSpec((3, 2 * cin, cout), lambda b, s: (0, 0, 0)),
            pl.BlockSpec((3, 2 * cin, cout), lambda b, s: (0, 0, 0)),
            pl.BlockSpec((1, cout), lambda b, s: (0, 0)),
            pl.BlockSpec((3, 3 * cout, cout), lambda b, s: (0, 0, 0)),
            pl.BlockSpec((1, cout), lambda b, s: (0, 0)),
        ],
        out_specs=pl.BlockSpec((1, c2 + cout, th, wu),
                               lambda b, s: (b, 0, s, 0)),
        compiler_params=pltpu.CompilerParams(
            dimension_semantics=("parallel", "parallel"),
            vmem_limit_bytes=48 * 2 ** 20),
    )(x1e, x2_nchw, w1a, w1b, b1r, w2s, b2r)

    return out


# speedup vs baseline: 2.8020x; 1.0215x over previous
import functools

import jax
import jax.numpy as jnp
from jax.experimental import pallas as pl
from jax.experimental.pallas import tpu as pltpu


def _block_kernel(x1e_ref, x2_ref, w1a_ref, w1b_ref, b1_ref, w2_ref, b2_ref,
                  o_ref, *, th, ws):
    cin = x1e_ref.shape[1]
    c2 = x2_ref.shape[1]
    cout = b1_ref.shape[1]

    s = pl.program_id(1)
    nlow = th // 2 + 2

    def eye(n):
        r = jax.lax.broadcasted_iota(jnp.int32, (n, n), 0)
        c = jax.lax.broadcasted_iota(jnp.int32, (n, n), 1)
        return (r == c).astype(jnp.bfloat16)

    xs = x1e_ref[0, :, pl.ds(s * (th // 2) * ws, nlow * ws)]
    low = jax.lax.dot_general(xs, eye(cin), (((0,), (0,)), ((), ())),
                              preferred_element_type=jnp.float32)
    low = low.astype(jnp.bfloat16).reshape(nlow, ws, cin)

    uph = jnp.concatenate([low[:, None], low[:, None]],
                          axis=1).reshape(2 * nlow, ws, cin)

    uphm1 = jnp.concatenate([uph[:, :1], uph[:, :ws - 1]], axis=1)
    uphp1 = jnp.concatenate([uph[:, 1:], uph[:, ws - 1:]], axis=1)

    ops1a = jnp.concatenate([uphm1, uph], axis=2)
    ops1b = jnp.concatenate([uph, uphp1], axis=2)

    def conv3(ops, w_ref, rows, k):
        acc = None
        for ky in range(3):
            a = ops[ky:ky + rows].reshape(rows * ws, k)
            p = jnp.dot(a, w_ref[ky], preferred_element_type=jnp.float32)
            acc = p if acc is None else acc + p
        return acc.reshape(rows, ws, -1)

    y1a = conv3(ops1a, w1a_ref, th + 2, 2 * cin)
    y1b = conv3(ops1b, w1b_ref, th + 2, 2 * cin)

    first = s == 0
    last = s == pl.num_programs(1) - 1
    y1a = jnp.where(first, jnp.concatenate([y1a[2:3], y1a[1:]], axis=0), y1a)
    y1b = jnp.where(first, jnp.concatenate([y1b[2:3], y1b[1:]], axis=0), y1b)
    y1a = jnp.where(last, jnp.concatenate([y1a[:th + 1], y1a[th - 1:th]],
                                          axis=0), y1a)
    y1b = jnp.where(last, jnp.concatenate([y1b[:th + 1], y1b[th - 1:th]],
                                          axis=0), y1b)

    b1v = b1_ref[...]
    a1a = jnp.maximum(y1a + b1v, 0.0).astype(jnp.bfloat16)
    a1b = jnp.maximum(y1b + b1v, 0.0).astype(jnp.bfloat16)

    a1bm1 = jnp.concatenate([a1b[:, :1], a1b[:, :ws - 1]], axis=1)
    a1ap1 = jnp.concatenate([a1a[:, 1:], a1a[:, ws - 1:]], axis=1)
    ops2a = jnp.concatenate([a1bm1, a1a, a1b], axis=2)
    ops2b = jnp.concatenate([a1a, a1b, a1ap1], axis=2)

    y2a = conv3(ops2a, w2_ref, th, 3 * cout)
    y2b = conv3(ops2b, w2_ref, th, 3 * cout)

    b2v = b2_ref[...]
    y2a = jnp.maximum(y2a + b2v, 0.0)
    y2b = jnp.maximum(y2b + b2v, 0.0)

    y2 = jnp.concatenate([y2a[:, :, None], y2b[:, :, None]],
                         axis=2).reshape(th * 2 * ws, cout)

    hi = y2.astype(jnp.bfloat16)
    lo = (y2 - hi.astype(jnp.float32)).astype(jnp.bfloat16)
    ic = eye(cout)
    dn = (((1,), (1,)), ((), ()))
    y2t = (jax.lax.dot_general(ic, hi, dn,
                               preferred_element_type=jnp.float32)
           + jax.lax.dot_general(ic, lo, dn,
                                 preferred_element_type=jnp.float32))

    o_ref[0, :c2] = x2_ref[0]
    o_ref[0, c2:] = y2t.reshape(cout, th, 2 * ws).astype(o_ref.dtype)


def _tile_rows(hu, cap=64):
    th = min(hu, cap)
    while hu % th != 0 or th % 2 != 0:
        th -= 1
    return max(th, 2)


def kernel(x1_nchw, x2_nchw, w1, b1, w2, b2):
    n, cin, hs, ws = x1_nchw.shape
    cout = w1.shape[-1]
    c2 = x2_nchw.shape[1]
    hu, wu = 2 * hs, 2 * ws

    th = _tile_rows(hu)
    n_strips = hu // th

    x1e = jnp.pad(x1_nchw.astype(jnp.bfloat16),
                  ((0, 0), (0, 0), (1, 1), (0, 0)), mode="edge")
    x1e = x1e.reshape(n, cin, (hs + 2) * ws)

    w1a = jnp.concatenate([w1[:, 0], w1[:, 1] + w1[:, 2]],
                          axis=1).astype(jnp.bfloat16)
    w1b = jnp.concatenate([w1[:, 0] + w1[:, 1], w1[:, 2]],
                          axis=1).astype(jnp.bfloat16)
    w2s = w2.reshape(3, 3 * cout, cout).astype(jnp.bfloat16)
    b1r = b1.reshape(1, cout)
    b2r = b2.reshape(1, cout)

    body = functools.partial(_block_kernel, th=th, ws=ws)

    out = pl.pallas_call(
        body,
        out_shape=jax.ShapeDtypeStruct((n, c2 + cout, hu, wu),
                                       x1_nchw.dtype),
        grid=(n, n_strips),
        in_specs=[
            pl.BlockSpec((1, cin, (hs + 2) * ws), lambda b, s: (b, 0, 0)),
            pl.BlockSpec((1, c2, th, wu), lambda b, s: (b, 0, s, 0)),
            pl.BlockSpec((3, 2 * cin, cout), lambda b, s: (0, 0, 0)),
            pl.BlockSpec((3, 2 * cin, cout), lambda b, s: (0, 0, 0)),
            pl.BlockSpec((1, cout), lambda b, s: (0, 0)),
            pl.BlockSpec((3, 3 * cout, cout), lambda b, s: (0, 0, 0)),
            pl.BlockSpec((1, cout), lambda b, s: (0, 0)),
        ],
        out_specs=pl.BlockSpec((1, c2 + cout, th, wu),
                               lambda b, s: (b, 0, s, 0)),
        compiler_params=pltpu.CompilerParams(
            dimension_semantics=("parallel", "parallel"),
            vmem_limit_bytes=48 * 2 ** 20),
    )(x1e, x2_nchw, w1a, w1b, b1r, w2s, b2r)

    return out


# speedup vs baseline: 3.2859x; 1.1727x over previous
import functools

import jax
import jax.numpy as jnp
from jax.experimental import pallas as pl
from jax.experimental.pallas import tpu as pltpu


def _block_kernel(x1e_ref, x2_ref, w1a_ref, w1b_ref, b1_ref, w2_ref, b2_ref,
                  o_ref, *, th, ws):
    cin = x1e_ref.shape[1]
    c2 = x2_ref.shape[1]
    cout = b1_ref.shape[1]

    s = pl.program_id(1)
    nlow = th // 2 + 2

    xs = x1e_ref[0, :, pl.ds(s * (th // 2) * ws, nlow * ws)]
    low = jnp.transpose(xs, (1, 0)).reshape(nlow, ws, cin)

    uph = jnp.concatenate([low[:, None], low[:, None]],
                          axis=1).reshape(2 * nlow, ws, cin)

    uphm1 = jnp.concatenate([uph[:, :1], uph[:, :ws - 1]], axis=1)
    uphp1 = jnp.concatenate([uph[:, 1:], uph[:, ws - 1:]], axis=1)

    ops1a = jnp.concatenate([uphm1, uph], axis=2)
    ops1b = jnp.concatenate([uph, uphp1], axis=2)

    def conv3(ops, w_ref, rows, k):
        acc = None
        for ky in range(3):
            a = ops[ky:ky + rows].reshape(rows * ws, k)
            p = jnp.dot(a, w_ref[ky], preferred_element_type=jnp.float32)
            acc = p if acc is None else acc + p
        return acc.reshape(rows, ws, -1)

    y1a = conv3(ops1a, w1a_ref, th + 2, 2 * cin)
    y1b = conv3(ops1b, w1b_ref, th + 2, 2 * cin)

    first = s == 0
    last = s == pl.num_programs(1) - 1
    y1a = jnp.where(first, jnp.concatenate([y1a[2:3], y1a[1:]], axis=0), y1a)
    y1b = jnp.where(first, jnp.concatenate([y1b[2:3], y1b[1:]], axis=0), y1b)
    y1a = jnp.where(last, jnp.concatenate([y1a[:th + 1], y1a[th - 1:th]],
                                          axis=0), y1a)
    y1b = jnp.where(last, jnp.concatenate([y1b[:th + 1], y1b[th - 1:th]],
                                          axis=0), y1b)

    b1v = b1_ref[...]
    a1a = jnp.maximum(y1a + b1v, 0.0).astype(jnp.bfloat16)
    a1b = jnp.maximum(y1b + b1v, 0.0).astype(jnp.bfloat16)

    a1bm1 = jnp.concatenate([a1b[:, :1], a1b[:, :ws - 1]], axis=1)
    a1ap1 = jnp.concatenate([a1a[:, 1:], a1a[:, ws - 1:]], axis=1)
    ops2a = jnp.concatenate([a1bm1, a1a, a1b], axis=2)
    ops2b = jnp.concatenate([a1a, a1b, a1ap1], axis=2)

    y2a = conv3(ops2a, w2_ref, th, 3 * cout)
    y2b = conv3(ops2b, w2_ref, th, 3 * cout)

    b2v = b2_ref[...]
    y2a = jnp.maximum(y2a + b2v, 0.0)
    y2b = jnp.maximum(y2b + b2v, 0.0)

    y2 = jnp.concatenate([y2a[:, :, None], y2b[:, :, None]],
                         axis=2).reshape(th, 2 * ws, cout)
    y2t = jnp.transpose(y2, (2, 0, 1))

    o_ref[0, :c2] = x2_ref[0]
    o_ref[0, c2:] = y2t.astype(o_ref.dtype)


def _tile_rows(hu, cap=64):
    th = min(hu, cap)
    while hu % th != 0 or th % 2 != 0:
        th -= 1
    return max(th, 2)


def kernel(x1_nchw, x2_nchw, w1, b1, w2, b2):
    n, cin, hs, ws = x1_nchw.shape
    cout = w1.shape[-1]
    c2 = x2_nchw.shape[1]
    hu, wu = 2 * hs, 2 * ws

    th = _tile_rows(hu)
    n_strips = hu // th

    x1e = jnp.pad(x1_nchw.astype(jnp.bfloat16),
                  ((0, 0), (0, 0), (1, 1), (0, 0)), mode="edge")
    x1e = x1e.reshape(n, cin, (hs + 2) * ws)

    w1a = jnp.concatenate([w1[:, 0], w1[:, 1] + w1[:, 2]],
                          axis=1).astype(jnp.bfloat16)
    w1b = jnp.concatenate([w1[:, 0] + w1[:, 1], w1[:, 2]],
                          axis=1).astype(jnp.bfloat16)
    w2s = w2.reshape(3, 3 * cout, cout).astype(jnp.bfloat16)
    b1r = b1.reshape(1, cout)
    b2r = b2.reshape(1, cout)

    body = functools.partial(_block_kernel, th=th, ws=ws)

    out = pl.pallas_call(
        body,
        out_shape=jax.ShapeDtypeStruct((n, c2 + cout, hu, wu),
                                       x1_nchw.dtype),
        grid=(n, n_strips),
        in_specs=[
            pl.BlockSpec((1, cin, (hs + 2) * ws), lambda b, s: (b, 0, 0)),
            pl.BlockSpec((1, c2, th, wu), lambda b, s: (b, 0, s, 0)),
            pl.BlockSpec((3, 2 * cin, cout), lambda b, s: (0, 0, 0)),
            pl.BlockSpec((3, 2 * cin, cout), lambda b, s: (0, 0, 0)),
            pl.BlockSpec((1, cout), lambda b, s: (0, 0)),
            pl.BlockSpec((3, 3 * cout, cout), lambda b, s: (0, 0, 0)),
            pl.BlockSpec((1, cout), lambda b, s: (0, 0)),
        ],
        out_specs=pl.BlockSpec((1, c2 + cout, th, wu),
                               lambda b, s: (b, 0, s, 0)),
        compiler_params=pltpu.CompilerParams(
            dimension_semantics=("parallel", "parallel"),
            vmem_limit_bytes=48 * 2 ** 20),
    )(x1e, x2_nchw, w1a, w1b, b1r, w2s, b2r)

    return out


# speedup vs baseline: 3.3327x; 1.0142x over previous
import functools

import jax
import jax.numpy as jnp
from jax.experimental import pallas as pl
from jax.experimental.pallas import tpu as pltpu


def _block_kernel(x1e_ref, x2_ref, w1a_ref, w1b_ref, b1_ref, w2_ref, b2_ref,
                  o_ref, *, th, ws):
    cin = x1e_ref.shape[1]
    c2 = x2_ref.shape[1]
    cout = b1_ref.shape[1]

    s = pl.program_id(1)
    nlow = th // 2 + 2

    xs = x1e_ref[0, :, pl.ds(s * (th // 2) * ws, nlow * ws)]
    low = jnp.transpose(xs, (1, 0)).reshape(nlow, ws, cin)

    uph = jnp.concatenate([low[:, None], low[:, None]],
                          axis=1).reshape(2 * nlow, ws, cin)

    uphm1 = jnp.concatenate([uph[:, :1], uph[:, :ws - 1]], axis=1)
    uphp1 = jnp.concatenate([uph[:, 1:], uph[:, ws - 1:]], axis=1)

    ops1a = jnp.concatenate([uphm1, uph], axis=2)
    ops1b = jnp.concatenate([uph, uphp1], axis=2)

    def conv3(ops, w_ref, rows, k):
        acc = None
        for ky in range(3):
            a = ops[ky:ky + rows].reshape(rows * ws, k)
            p = jnp.dot(a, w_ref[ky], preferred_element_type=jnp.float32)
            acc = p if acc is None else acc + p
        return acc.reshape(rows, ws, -1)

    y1a = conv3(ops1a, w1a_ref, th + 2, 2 * cin)
    y1b = conv3(ops1b, w1b_ref, th + 2, 2 * cin)

    first = s == 0
    last = s == pl.num_programs(1) - 1
    y1a = jnp.where(first, jnp.concatenate([y1a[2:3], y1a[1:]], axis=0), y1a)
    y1b = jnp.where(first, jnp.concatenate([y1b[2:3], y1b[1:]], axis=0), y1b)
    y1a = jnp.where(last, jnp.concatenate([y1a[:th + 1], y1a[th - 1:th]],
                                          axis=0), y1a)
    y1b = jnp.where(last, jnp.concatenate([y1b[:th + 1], y1b[th - 1:th]],
                                          axis=0), y1b)

    b1v = b1_ref[...]
    a1a = jnp.maximum(y1a + b1v, 0.0).astype(jnp.bfloat16)
    a1b = jnp.maximum(y1b + b1v, 0.0).astype(jnp.bfloat16)

    a1bm1 = jnp.concatenate([a1b[:, :1], a1b[:, :ws - 1]], axis=1)
    a1ap1 = jnp.concatenate([a1a[:, 1:], a1a[:, ws - 1:]], axis=1)
    ops2a = jnp.concatenate([a1bm1, a1a, a1b], axis=2)
    ops2b = jnp.concatenate([a1a, a1b, a1ap1], axis=2)

    y2a = conv3(ops2a, w2_ref, th, 3 * cout)
    y2b = conv3(ops2b, w2_ref, th, 3 * cout)

    b2v = b2_ref[...]
    y2a = jnp.maximum(y2a + b2v, 0.0)
    y2b = jnp.maximum(y2b + b2v, 0.0)

    y2 = jnp.concatenate([y2a[:, :, None], y2b[:, :, None]],
                         axis=2).reshape(th, 2 * ws, cout)
    y2t = jnp.transpose(y2.astype(jnp.bfloat16), (2, 0, 1))

    o_ref[0, :c2] = x2_ref[0]
    o_ref[0, c2:] = y2t.astype(o_ref.dtype)


def _tile_rows(hu, cap=64):
    th = min(hu, cap)
    while hu % th != 0 or th % 2 != 0:
        th -= 1
    return max(th, 2)


def kernel(x1_nchw, x2_nchw, w1, b1, w2, b2):
    n, cin, hs, ws = x1_nchw.shape
    cout = w1.shape[-1]
    c2 = x2_nchw.shape[1]
    hu, wu = 2 * hs, 2 * ws

    th = _tile_rows(hu)
    n_strips = hu // th

    x1e = jnp.pad(x1_nchw.astype(jnp.bfloat16),
                  ((0, 0), (0, 0), (1, 1), (0, 0)), mode="edge")
    x1e = x1e.reshape(n, cin, (hs + 2) * ws)

    w1a = jnp.concatenate([w1[:, 0], w1[:, 1] + w1[:, 2]],
                          axis=1).astype(jnp.bfloat16)
    w1b = jnp.concatenate([w1[:, 0] + w1[:, 1], w1[:, 2]],
                          axis=1).astype(jnp.bfloat16)
    w2s = w2.reshape(3, 3 * cout, cout).astype(jnp.bfloat16)
    b1r = b1.reshape(1, cout)
    b2r = b2.reshape(1, cout)

    body = functools.partial(_block_kernel, th=th, ws=ws)

    out = pl.pallas_call(
        body,
        out_shape=jax.ShapeDtypeStruct((n, c2 + cout, hu, wu),
                                       x1_nchw.dtype),
        grid=(n, n_strips),
        in_specs=[
            pl.BlockSpec((1, cin, (hs + 2) * ws), lambda b, s: (b, 0, 0)),
            pl.BlockSpec((1, c2, th, wu), lambda b, s: (b, 0, s, 0)),
            pl.BlockSpec((3, 2 * cin, cout), lambda b, s: (0, 0, 0)),
            pl.BlockSpec((3, 2 * cin, cout), lambda b, s: (0, 0, 0)),
            pl.BlockSpec((1, cout), lambda b, s: (0, 0)),
            pl.BlockSpec((3, 3 * cout, cout), lambda b, s: (0, 0, 0)),
            pl.BlockSpec((1, cout), lambda b, s: (0, 0)),
        ],
        out_specs=pl.BlockSpec((1, c2 + cout, th, wu),
                               lambda b, s: (b, 0, s, 0)),
        compiler_params=pltpu.CompilerParams(
            dimension_semantics=("parallel", "parallel"),
            vmem_limit_bytes=48 * 2 ** 20),
    )(x1e, x2_nchw, w1a, w1b, b1r, w2s, b2r)

    return out
